# baseline (speedup 1.0000x reference)
import jax
import jax.numpy as jnp
from jax import lax
from jax.experimental import pallas as pl
from jax.experimental.pallas import tpu as pltpu
from jax.experimental.pallas import tpu_sc as plsc

HIST = 200
DP = 64
NLANE = 16
NGRP = DP // NLANE


def _body(x0_hbm, x1_hbm, emb1_hbm, lin_w_hbm, lin_b_hbm, emb2_hbm,
          out_hbm,
          idx_v, rows_v, x1_v, y2_v, linw_v, linb_v, out_v,
          sem_r, sem_y, sem_w):
    cid = lax.axis_index("c")
    sid = lax.axis_index("s")

    @pl.when(jnp.logical_and(cid == 0, sid == 0))
    def _():
        pltpu.sync_copy(x0_hbm, idx_v)
        pltpu.sync_copy(x1_hbm, x1_v)

        g0 = pltpu.async_copy(emb1_hbm.at[idx_v.at[0]],
                              rows_v.at[pl.ds(0, HIST // 2)], sem_r)
        g1 = pltpu.async_copy(emb1_hbm.at[idx_v.at[1]],
                              rows_v.at[pl.ds(HIST // 2, HIST // 2)], sem_r)
        gy = pltpu.async_copy(emb2_hbm.at[x1_v], y2_v, sem_y)
        gw = pltpu.async_copy(lin_w_hbm, linw_v, sem_w)
        gb = pltpu.async_copy(lin_b_hbm, linb_v, sem_w)

        gy.wait()
        gw.wait()
        gb.wait()

        y2g = [y2_v[0, pl.ds(j * NLANE, NLANE)] for j in range(NGRP)]
        v = [jnp.zeros((NLANE,), jnp.float32) for _ in range(NGRP)]
        for d in range(DP):
            bd = lax.broadcast_in_dim(y2g[d // NLANE][d % NLANE],
                                      (NLANE,), ())
            for j in range(NGRP):
                v[j] = v[j] + bd * linw_v[d, pl.ds(j * NLANE, NLANE)]

        g0.wait()
        g1.wait()

        def row_add(i, acc):
            return tuple(acc[j] + rows_v[i, pl.ds(j * NLANE, NLANE)]
                         for j in range(NGRP))

        s = lax.fori_loop(
            0, HIST, row_add,
            tuple(jnp.zeros((NLANE,), jnp.float32) for _ in range(NGRP)))

        acc = jnp.zeros((NLANE,), jnp.float32)
        inv_n = 1.0 / HIST
        for j in range(NGRP):
            acc = acc + (s[j] * inv_n) * v[j]
            acc = acc + y2g[j] * linb_v[pl.ds(j * NLANE, NLANE)]

        total = acc[0]
        for i in range(1, NLANE):
            total = total + acc[i]
        tb = lax.broadcast_in_dim(total, (NLANE,), ())
        out_v[...] = 1.0 / (1.0 + jnp.exp(-tb))
        pltpu.sync_copy(out_v, out_hbm)


def kernel(x0, x1, emb1_w, lin_w, lin_b, emb2_w):
    x0_2d = x0.astype(jnp.int32).reshape(2, HIST // 2)
    x1_i = x1.astype(jnp.int32)

    mesh = plsc.VectorSubcoreMesh(core_axis_name="c", subcore_axis_name="s")
    run = pl.kernel(
        _body,
        out_type=jax.ShapeDtypeStruct((NLANE,), jnp.float32),
        mesh=mesh,
        compiler_params=pltpu.CompilerParams(
            use_tc_tiling_on_sc=False, needs_layout_passes=False),
        scratch_types=[
            pltpu.VMEM((2, HIST // 2), jnp.int32),
            pltpu.VMEM((HIST, DP), jnp.float32),
            pltpu.VMEM((1,), jnp.int32),
            pltpu.VMEM((1, DP), jnp.float32),
            pltpu.VMEM((DP, DP), jnp.float32),
            pltpu.VMEM((DP,), jnp.float32),
            pltpu.VMEM((NLANE,), jnp.float32),
            pltpu.SemaphoreType.DMA,
            pltpu.SemaphoreType.DMA,
            pltpu.SemaphoreType.DMA,
        ],
    )
    out = run(x0_2d, x1_i, emb1_w, lin_w, lin_b, emb2_w)
    return out[0]
